# Initial kernel scaffold; baseline (speedup 1.0000x reference)
import jax
import jax.numpy as jnp
from jax.experimental import pallas as pl


def kernel(c0, h0, c1, h1, instruction_pointer, current_step, node_embeddings, edge_sources, edge_dests, edge_types, true_indexes, false_indexes, raise_indexes, exit_node_indexes, raise_node_indexes, step_limits, Wx0, Wh0, b0, Wx1, Wh1, b1, Wr, br, Wb, bb):
    raise NotImplementedError("write your pallas kernel here")


# speedup vs baseline: 11.6129x; 11.6129x over previous
import functools

import jax
import jax.numpy as jnp
from jax.experimental import pallas as pl
from jax.experimental.pallas import tpu as pltpu

B, N, H = 32, 512, 256


def _fused_body(cs_ref, sl_ref, ex_ref, rz_ref,
                c0_ref, h0_ref, c1_ref, h1_ref, ne_ref, ip_ref,
                ti_ref, fi_ref, ri_ref,
                wx0_ref, wh0_ref, b0_ref, wx1_ref, wh1_ref, b1_ref,
                wr_ref, wb_ref, rb_ref,
                out_ref):
    b = pl.program_id(0)
    c0 = c0_ref[0]
    h0 = h0_ref[0]
    c1 = c1_ref[0]
    h1 = h1_ref[0]
    ne = ne_ref[0]

    dot = functools.partial(jnp.dot, preferred_element_type=jnp.float32)

    z0 = dot(ne, wx0_ref[...]) + dot(h0, wh0_ref[...]) + b0_ref[...]
    i0 = jax.nn.sigmoid(z0[:, 0:H])
    f0 = jax.nn.sigmoid(z0[:, H:2 * H])
    g0 = jnp.tanh(z0[:, 2 * H:3 * H])
    o0 = jax.nn.sigmoid(z0[:, 3 * H:4 * H])
    c0n = f0 * c0 + i0 * g0
    h0n = o0 * jnp.tanh(c0n)

    z1 = dot(h0n, wx1_ref[...]) + dot(h1, wh1_ref[...]) + b1_ref[...]
    i1 = jax.nn.sigmoid(z1[:, 0:H])
    f1 = jax.nn.sigmoid(z1[:, H:2 * H])
    g1 = jnp.tanh(z1[:, 2 * H:3 * H])
    o1 = jax.nn.sigmoid(z1[:, 3 * H:4 * H])
    c1n = f1 * c1 + i1 * g1
    h1n = o1 * jnp.tanh(c1n)

    ex = ex_ref[b]
    rz = rz_ref[b]
    row = jax.lax.broadcasted_iota(jnp.int32, (N, 1), 0)
    frozen = (row == ex) | (row == rz)
    c0n = jnp.where(frozen, c0, c0n)
    h0n = jnp.where(frozen, h0, h0n)
    c1n = jnp.where(frozen, c1, c1n)
    h1n = jnp.where(frozen, h1, h1n)

    wr = wr_ref[...]
    wb = wb_ref[...]
    dr = (jnp.sum(c0n * wr[:, 0:H], axis=1, keepdims=True)
          + jnp.sum(h0n * wr[:, H:2 * H], axis=1, keepdims=True)
          + jnp.sum(c1n * wr[:, 2 * H:3 * H], axis=1, keepdims=True)
          + jnp.sum(h1n * wr[:, 3 * H:4 * H], axis=1, keepdims=True)
          + rb_ref[0, 0])
    db = (jnp.sum(c0n * wb[:, 0:H], axis=1, keepdims=True)
          + jnp.sum(h0n * wb[:, H:2 * H], axis=1, keepdims=True)
          + jnp.sum(c1n * wb[:, 2 * H:3 * H], axis=1, keepdims=True)
          + jnp.sum(h1n * wb[:, 3 * H:4 * H], axis=1, keepdims=True)
          + rb_ref[0, 1])
    p_raise = jax.nn.sigmoid(dr)
    p_raise = jnp.where(row == ex, 0.0, p_raise)
    p_noraise = 1.0 - p_raise
    p_true = p_noraise * jax.nn.sigmoid(db)
    p_false = p_noraise - p_true

    ipc = ip_ref[0]
    wt = p_true * ipc
    wf = p_false * ipc
    wrs = p_raise * ipc

    col = jax.lax.broadcasted_iota(jnp.int32, (N, N), 1)
    ti = ti_ref[0]
    fi = fi_ref[0]
    ri = ri_ref[0]
    a = (jnp.where(col == ti, wt, 0.0)
         + jnp.where(col == fi, wf, 0.0)
         + jnp.where(col == ri, wrs, 0.0))

    dimn = (((0,), (0,)), ((), ()))
    ip_new = jax.lax.dot_general(a, jnp.ones((N, 1), jnp.float32),
                                 dimension_numbers=dimn,
                                 preferred_element_type=jnp.float32)
    inv = 1.0 / (ip_new + 1e-07)

    def agg(x):
        return jax.lax.dot_general(a, x, dimension_numbers=dimn,
                                   preferred_element_type=jnp.float32) * inv

    not_done = cs_ref[b] < sl_ref[b]
    out_ref[0, :, 0:H] = jnp.where(not_done, agg(c0n), c0)
    out_ref[0, :, H:2 * H] = jnp.where(not_done, agg(h0n), h0)
    out_ref[0, :, 2 * H:3 * H] = jnp.where(not_done, agg(c1n), c1)
    out_ref[0, :, 3 * H:4 * H] = jnp.where(not_done, agg(h1n), h1)
    out_ref[0, :, 4 * H:4 * H + 1] = jnp.where(not_done, ip_new, ipc)


def kernel(c0, h0, c1, h1, instruction_pointer, current_step,
           node_embeddings, edge_sources, edge_dests, edge_types,
           true_indexes, false_indexes, raise_indexes, exit_node_indexes,
           raise_node_indexes, step_limits, Wx0, Wh0, b0, Wx1, Wh1, b1,
           Wr, br, Wb, bb):
    del edge_sources, edge_dests, edge_types

    ip = instruction_pointer.reshape(B, N, 1)
    ti = true_indexes.reshape(B, N, 1)
    fi = false_indexes.reshape(B, N, 1)
    ri = raise_indexes.reshape(B, N, 1)
    b0r = b0.reshape(1, 4 * H)
    b1r = b1.reshape(1, 4 * H)
    wr = (Wr[:, 0] - Wr[:, 1]).reshape(1, 4 * H)
    wb = (Wb[:, 0] - Wb[:, 1]).reshape(1, 4 * H)
    rb = jnp.stack([br[0] - br[1], bb[0] - bb[1]]).reshape(1, 2)

    bspec = lambda shape: pl.BlockSpec(shape, lambda *_: (0,) * len(shape))
    batched = lambda shape: pl.BlockSpec((1,) + shape,
                                         lambda b_, *_: (b_,) + (0,) * len(shape))

    grid_spec = pltpu.PrefetchScalarGridSpec(
        num_scalar_prefetch=4,
        grid=(B,),
        in_specs=[
            batched((N, H)), batched((N, H)), batched((N, H)),
            batched((N, H)), batched((N, H)),
            batched((N, 1)),
            batched((N, 1)), batched((N, 1)), batched((N, 1)),
            bspec((H, 4 * H)), bspec((H, 4 * H)), bspec((1, 4 * H)),
            bspec((H, 4 * H)), bspec((H, 4 * H)), bspec((1, 4 * H)),
            bspec((1, 4 * H)), bspec((1, 4 * H)), bspec((1, 2)),
        ],
        out_specs=batched((N, 4 * H + 1)),
    )

    out = pl.pallas_call(
        _fused_body,
        grid_spec=grid_spec,
        out_shape=jax.ShapeDtypeStruct((B, N, 4 * H + 1), jnp.float32),
    )(current_step, step_limits, exit_node_indexes, raise_node_indexes,
      c0, h0, c1, h1, node_embeddings, ip, ti, fi, ri,
      Wx0, Wh0, b0r, Wx1, Wh1, b1r, wr, wb, rb)
    return out


# speedup vs baseline: 13.3274x; 1.1476x over previous
import functools

import jax
import jax.numpy as jnp
from jax.experimental import pallas as pl
from jax.experimental.pallas import tpu as pltpu

B, N, H = 32, 512, 256
CW = 4 * H + 128


def _fused_body(cs_ref, sl_ref, ex_ref, rz_ref,
                c0_ref, h0_ref, c1_ref, h1_ref, ne_ref, ip_ref,
                ti_ref, fi_ref, ri_ref,
                wx0_ref, wh0_ref, b0_ref, wx1_ref, wh1_ref, b1_ref,
                wrb_ref, rb_ref,
                out_ref, cat_ref):
    b = pl.program_id(0)
    c0 = c0_ref[0]
    h0 = h0_ref[0]
    c1 = c1_ref[0]
    h1 = h1_ref[0]
    ne = ne_ref[0]

    dot = functools.partial(jnp.dot, preferred_element_type=jnp.float32)

    z0 = dot(ne, wx0_ref[...]) + dot(h0, wh0_ref[...]) + b0_ref[...]
    i0 = jax.nn.sigmoid(z0[:, 0:H])
    f0 = jax.nn.sigmoid(z0[:, H:2 * H])
    g0 = jnp.tanh(z0[:, 2 * H:3 * H])
    o0 = jax.nn.sigmoid(z0[:, 3 * H:4 * H])
    c0n = f0 * c0 + i0 * g0
    h0n = o0 * jnp.tanh(c0n)

    z1 = dot(h0n, wx1_ref[...]) + dot(h1, wh1_ref[...]) + b1_ref[...]
    i1 = jax.nn.sigmoid(z1[:, 0:H])
    f1 = jax.nn.sigmoid(z1[:, H:2 * H])
    g1 = jnp.tanh(z1[:, 2 * H:3 * H])
    o1 = jax.nn.sigmoid(z1[:, 3 * H:4 * H])
    c1n = f1 * c1 + i1 * g1
    h1n = o1 * jnp.tanh(c1n)

    ex = ex_ref[b]
    rz = rz_ref[b]
    row = jax.lax.broadcasted_iota(jnp.int32, (N, 1), 0)
    frozen = (row == ex) | (row == rz)
    cat_ref[:, 0:H] = jnp.where(frozen, c0, c0n)
    cat_ref[:, H:2 * H] = jnp.where(frozen, h0, h0n)
    cat_ref[:, 2 * H:3 * H] = jnp.where(frozen, c1, c1n)
    cat_ref[:, 3 * H:4 * H] = jnp.where(frozen, h1, h1n)
    cat_ref[:, 4 * H:CW] = jnp.ones((N, 128), jnp.float32)
    cat = cat_ref[...]

    logits = dot(cat[:, 0:4 * H], wrb_ref[...])
    dr_row = logits[:, 0:1].T + rb_ref[0, 0]
    db_row = logits[:, 1:2].T + rb_ref[0, 1]
    lane = jax.lax.broadcasted_iota(jnp.int32, (1, N), 1)
    p_raise = jax.nn.sigmoid(dr_row)
    p_raise = jnp.where(lane == ex, 0.0, p_raise)
    p_noraise = 1.0 - p_raise
    p_true = p_noraise * jax.nn.sigmoid(db_row)
    p_false = p_noraise - p_true

    ipr = ip_ref[0]
    wt = p_true * ipr
    wf = p_false * ipr
    wrs = p_raise * ipr

    rowd = jax.lax.broadcasted_iota(jnp.int32, (N, N), 0)
    at = (jnp.where(rowd == ti_ref[0], wt, 0.0)
          + jnp.where(rowd == fi_ref[0], wf, 0.0)
          + jnp.where(rowd == ri_ref[0], wrs, 0.0))

    agg = dot(at, cat)
    ip_new = agg[:, 4 * H:4 * H + 1]
    inv = 1.0 / (ip_new + 1e-07)

    not_done = cs_ref[b] < sl_ref[b]
    out_ref[0, :, 0:H] = jnp.where(not_done, agg[:, 0:H] * inv, c0)
    out_ref[0, :, H:2 * H] = jnp.where(not_done, agg[:, H:2 * H] * inv, h0)
    out_ref[0, :, 2 * H:3 * H] = jnp.where(not_done, agg[:, 2 * H:3 * H] * inv, c1)
    out_ref[0, :, 3 * H:4 * H] = jnp.where(not_done, agg[:, 3 * H:4 * H] * inv, h1)
    out_ref[0, :, 4 * H:4 * H + 1] = jnp.where(not_done, ip_new, ipr.T)


def kernel(c0, h0, c1, h1, instruction_pointer, current_step,
           node_embeddings, edge_sources, edge_dests, edge_types,
           true_indexes, false_indexes, raise_indexes, exit_node_indexes,
           raise_node_indexes, step_limits, Wx0, Wh0, b0, Wx1, Wh1, b1,
           Wr, br, Wb, bb):
    del edge_sources, edge_dests, edge_types

    ip = instruction_pointer.reshape(B, 1, N)
    ti = true_indexes.reshape(B, 1, N)
    fi = false_indexes.reshape(B, 1, N)
    ri = raise_indexes.reshape(B, 1, N)
    b0r = b0.reshape(1, 4 * H)
    b1r = b1.reshape(1, 4 * H)
    wrb = jnp.stack([Wr[:, 0] - Wr[:, 1], Wb[:, 0] - Wb[:, 1]], axis=1)
    rb = jnp.stack([br[0] - br[1], bb[0] - bb[1]]).reshape(1, 2)

    bspec = lambda shape: pl.BlockSpec(shape, lambda *_: (0,) * len(shape))
    batched = lambda shape: pl.BlockSpec((1,) + shape,
                                         lambda b_, *_: (b_,) + (0,) * len(shape))

    grid_spec = pltpu.PrefetchScalarGridSpec(
        num_scalar_prefetch=4,
        grid=(B,),
        in_specs=[
            batched((N, H)), batched((N, H)), batched((N, H)),
            batched((N, H)), batched((N, H)),
            batched((1, N)),
            batched((1, N)), batched((1, N)), batched((1, N)),
            bspec((H, 4 * H)), bspec((H, 4 * H)), bspec((1, 4 * H)),
            bspec((H, 4 * H)), bspec((H, 4 * H)), bspec((1, 4 * H)),
            bspec((4 * H, 2)), bspec((1, 2)),
        ],
        out_specs=batched((N, 4 * H + 1)),
        scratch_shapes=[pltpu.VMEM((N, CW), jnp.float32)],
    )

    out = pl.pallas_call(
        _fused_body,
        grid_spec=grid_spec,
        out_shape=jax.ShapeDtypeStruct((B, N, 4 * H + 1), jnp.float32),
    )(current_step, step_limits, exit_node_indexes, raise_node_indexes,
      c0, h0, c1, h1, node_embeddings, ip, ti, fi, ri,
      Wx0, Wh0, b0r, Wx1, Wh1, b1r, wrb, rb)
    return out


# speedup vs baseline: 13.3500x; 1.0017x over previous
import functools

import jax
import jax.numpy as jnp
from jax.experimental import pallas as pl
from jax.experimental.pallas import tpu as pltpu

B, N, H = 32, 512, 256
CW = 4 * H + 128


def _fused_body(cs_ref, sl_ref, ex_ref, rz_ref,
                c0_ref, h0_ref, c1_ref, h1_ref, ne_ref, ip_ref,
                ti_ref, fi_ref, ri_ref,
                wx0_ref, wh0_ref, b0_ref, wx1_ref, wh1_ref, b1_ref,
                wrb_ref, rb_ref,
                out_ref, cat_ref):
    b = pl.program_id(0)
    c0 = c0_ref[0]
    h0 = h0_ref[0]
    c1 = c1_ref[0]
    h1 = h1_ref[0]
    ne = ne_ref[0]

    dot = functools.partial(jnp.dot, preferred_element_type=jnp.float32)

    z0 = dot(ne, wx0_ref[...]) + dot(h0, wh0_ref[...]) + b0_ref[...]
    i0 = jax.nn.sigmoid(z0[:, 0:H])
    f0 = jax.nn.sigmoid(z0[:, H:2 * H])
    g0 = jnp.tanh(z0[:, 2 * H:3 * H])
    o0 = jax.nn.sigmoid(z0[:, 3 * H:4 * H])
    c0n = f0 * c0 + i0 * g0
    h0n = o0 * jnp.tanh(c0n)

    z1 = dot(h0n, wx1_ref[...]) + dot(h1, wh1_ref[...]) + b1_ref[...]
    i1 = jax.nn.sigmoid(z1[:, 0:H])
    f1 = jax.nn.sigmoid(z1[:, H:2 * H])
    g1 = jnp.tanh(z1[:, 2 * H:3 * H])
    o1 = jax.nn.sigmoid(z1[:, 3 * H:4 * H])
    c1n = f1 * c1 + i1 * g1
    h1n = o1 * jnp.tanh(c1n)

    ex = ex_ref[b]
    rz = rz_ref[b]
    row = jax.lax.broadcasted_iota(jnp.int32, (N, 1), 0)
    frozen = (row == ex) | (row == rz)
    cat_ref[:, 0:H] = jnp.where(frozen, c0, c0n)
    cat_ref[:, H:2 * H] = jnp.where(frozen, h0, h0n)
    cat_ref[:, 2 * H:3 * H] = jnp.where(frozen, c1, c1n)
    cat_ref[:, 3 * H:4 * H] = jnp.where(frozen, h1, h1n)
    cat_ref[:, 4 * H:CW] = jnp.ones((N, 128), jnp.float32)
    cat = cat_ref[...]

    logits = dot(cat[:, 0:4 * H], wrb_ref[...])
    dr_row = logits[:, 0:1].T + rb_ref[0, 0]
    db_row = logits[:, 1:2].T + rb_ref[0, 1]
    lane = jax.lax.broadcasted_iota(jnp.int32, (1, N), 1)
    p_raise = jax.nn.sigmoid(dr_row)
    p_raise = jnp.where(lane == ex, 0.0, p_raise)
    p_noraise = 1.0 - p_raise
    p_true = p_noraise * jax.nn.sigmoid(db_row)
    p_false = p_noraise - p_true

    ipr = ip_ref[0]
    wt = p_true * ipr
    wf = p_false * ipr
    wrs = p_raise * ipr

    rowd = jax.lax.broadcasted_iota(jnp.int32, (N, N), 0)
    at = (jnp.where(rowd == ti_ref[0], wt, 0.0)
          + jnp.where(rowd == fi_ref[0], wf, 0.0)
          + jnp.where(rowd == ri_ref[0], wrs, 0.0))

    agg = dot(at, cat)
    ip_new = agg[:, 4 * H:4 * H + 1]
    inv = 1.0 / (ip_new + 1e-07)

    not_done = cs_ref[b] < sl_ref[b]
    out_ref[0, :, 0:H] = jnp.where(not_done, agg[:, 0:H] * inv, c0)
    out_ref[0, :, H:2 * H] = jnp.where(not_done, agg[:, H:2 * H] * inv, h0)
    out_ref[0, :, 2 * H:3 * H] = jnp.where(not_done, agg[:, 2 * H:3 * H] * inv, c1)
    out_ref[0, :, 3 * H:4 * H] = jnp.where(not_done, agg[:, 3 * H:4 * H] * inv, h1)
    out_ref[0, :, 4 * H:4 * H + 1] = jnp.where(not_done, ip_new, ipr.T)


def kernel(c0, h0, c1, h1, instruction_pointer, current_step,
           node_embeddings, edge_sources, edge_dests, edge_types,
           true_indexes, false_indexes, raise_indexes, exit_node_indexes,
           raise_node_indexes, step_limits, Wx0, Wh0, b0, Wx1, Wh1, b1,
           Wr, br, Wb, bb):
    del edge_sources, edge_dests, edge_types

    ip = instruction_pointer.reshape(B, 1, N)
    ti = true_indexes.reshape(B, 1, N)
    fi = false_indexes.reshape(B, 1, N)
    ri = raise_indexes.reshape(B, 1, N)
    b0r = b0.reshape(1, 4 * H)
    b1r = b1.reshape(1, 4 * H)
    wrb = jnp.stack([Wr[:, 0] - Wr[:, 1], Wb[:, 0] - Wb[:, 1]], axis=1)
    rb = jnp.stack([br[0] - br[1], bb[0] - bb[1]]).reshape(1, 2)

    bspec = lambda shape: pl.BlockSpec(shape, lambda *_: (0,) * len(shape))
    batched = lambda shape: pl.BlockSpec((1,) + shape,
                                         lambda b_, *_: (b_,) + (0,) * len(shape))

    grid_spec = pltpu.PrefetchScalarGridSpec(
        num_scalar_prefetch=4,
        grid=(B,),
        in_specs=[
            batched((N, H)), batched((N, H)), batched((N, H)),
            batched((N, H)), batched((N, H)),
            batched((1, N)),
            batched((1, N)), batched((1, N)), batched((1, N)),
            bspec((H, 4 * H)), bspec((H, 4 * H)), bspec((1, 4 * H)),
            bspec((H, 4 * H)), bspec((H, 4 * H)), bspec((1, 4 * H)),
            bspec((4 * H, 2)), bspec((1, 2)),
        ],
        out_specs=batched((N, 4 * H + 1)),
        scratch_shapes=[pltpu.VMEM((N, CW), jnp.float32)],
    )

    out = pl.pallas_call(
        _fused_body,
        grid_spec=grid_spec,
        out_shape=jax.ShapeDtypeStruct((B, N, 4 * H + 1), jnp.float32),
        compiler_params=pltpu.CompilerParams(
            dimension_semantics=("parallel",),
            vmem_limit_bytes=100 * 1024 * 1024,
        ),
    )(current_step, step_limits, exit_node_indexes, raise_node_indexes,
      c0, h0, c1, h1, node_embeddings, ip, ti, fi, ri,
      Wx0, Wh0, b0r, Wx1, Wh1, b1r, wrb, rb)
    return out
